# baseline (speedup 1.0000x reference)
import functools

import jax
import jax.numpy as jnp
from jax import lax
from jax.experimental import pallas as pl
from jax.experimental.pallas import tpu as pltpu
from jax.experimental.pallas import tpu_sc as plsc

B = 16384
D = 16
NC = 2
NS = 16
NW = NC * NS
BPW = B // NW
CHUNK = 128
NCHUNK = BPW // CHUNK
GROUP = 128 // D
BROW = 100096 // 128

_LOG1P_COEFS = (
    5.2394028005e-09,
    9.9999891058e-01,
    -4.9996224452e-01,
    3.3281842540e-01,
    -2.4635660618e-01,
    1.8468848463e-01,
    -1.2526661441e-01,
    6.6512479382e-02,
    -2.3038279977e-02,
    3.7526242255e-03,
)


def _softplus(x):
    t = jnp.exp(-jnp.abs(x))
    p = jnp.full((16,), _LOG1P_COEFS[-1], jnp.float32)
    for c in reversed(_LOG1P_COEFS[:-1]):
        p = p * t + jnp.float32(c)
    return jnp.maximum(x, jnp.float32(0.0)) + p


def _mirt_body(user_hbm, item_hbm, theta_hbm, a_hbm, b_hbm, out_hbm,
               uidx_v, iidx_v, urow_v, uoff_v, irow_v, ioff_v, brow_v, boff_v,
               th_v, a_v, b_v, out_v, sem_t, sem_a, sem_b):
    wid = lax.axis_index("s") * NC + lax.axis_index("c")

    pltpu.sync_copy(user_hbm.at[wid], uidx_v)
    pltpu.sync_copy(item_hbm.at[wid], iidx_v)

    iota16 = lax.iota(jnp.int32, 16)

    for k in range(NCHUNK):
        kvec = jnp.full((16,), k, jnp.int32)
        for j in range(CHUNK // 16):
            cols = j * 16 + iota16
            u = plsc.load_gather(uidx_v, [k * CHUNK + cols])
            i = plsc.load_gather(iidx_v, [k * CHUNK + cols])
            plsc.store_scatter(urow_v, [kvec, cols],
                               lax.shift_right_logical(u, 3))
            plsc.store_scatter(uoff_v, [kvec, cols],
                               lax.shift_left(jnp.bitwise_and(u, 7), 4))
            plsc.store_scatter(irow_v, [kvec, cols],
                               lax.shift_right_logical(i, 3))
            plsc.store_scatter(ioff_v, [kvec, cols],
                               lax.shift_left(jnp.bitwise_and(i, 7), 4))
            plsc.store_scatter(brow_v, [kvec, cols],
                               lax.shift_right_logical(i, 7))
            plsc.store_scatter(boff_v, [kvec, cols],
                               jnp.bitwise_and(i, 127))

    def fire(k, s):
        return (
            pltpu.async_copy(theta_hbm.at[urow_v.at[k]], th_v.at[s], sem_t),
            pltpu.async_copy(a_hbm.at[irow_v.at[k]], a_v.at[s], sem_a),
            pltpu.async_copy(b_hbm.at[brow_v.at[k]], b_v.at[s], sem_b),
        )

    inflight = fire(0, 0)
    for k in range(NCHUNK):
        s = k % 2
        cur = inflight
        if k + 1 < NCHUNK:
            inflight = fire(k + 1, (k + 1) % 2)
        for cp in cur:
            cp.wait()

        svec = jnp.full((16,), s, jnp.int32)
        kvec = jnp.full((16,), k, jnp.int32)

        def block(j, carry, k=k, s=s, svec=svec, kvec=kvec):
            rows = j * 16 + iota16
            offu = plsc.load_gather(uoff_v, [kvec, rows])
            offi = plsc.load_gather(ioff_v, [kvec, rows])
            offb = plsc.load_gather(boff_v, [kvec, rows])
            acc = -plsc.load_gather(b_v, [svec, rows, offb])
            for d in range(D):
                dv = jnp.full((16,), d, jnp.int32)
                th = plsc.load_gather(th_v, [svec, rows, offu + dv])
                av = plsc.load_gather(a_v, [svec, rows, offi + dv])
                acc = acc + _softplus(av) * th
            res = jnp.float32(1.0) / (jnp.float32(1.0) + jnp.exp(-acc))
            out_v[pl.ds(k * CHUNK + j * 16, 16)] = res
            return carry

        lax.fori_loop(0, CHUNK // 16, block, 0)

    pltpu.sync_copy(out_v, out_hbm.at[wid])


_mirt = functools.partial(
    pl.kernel,
    out_type=jax.ShapeDtypeStruct((NW, BPW), jnp.float32),
    mesh=plsc.VectorSubcoreMesh(core_axis_name="c", subcore_axis_name="s"),
    compiler_params=pltpu.CompilerParams(needs_layout_passes=False),
    scratch_types=[
        pltpu.VMEM((BPW,), jnp.int32),
        pltpu.VMEM((BPW,), jnp.int32),
        pltpu.VMEM((NCHUNK, CHUNK), jnp.int32),
        pltpu.VMEM((NCHUNK, CHUNK), jnp.int32),
        pltpu.VMEM((NCHUNK, CHUNK), jnp.int32),
        pltpu.VMEM((NCHUNK, CHUNK), jnp.int32),
        pltpu.VMEM((NCHUNK, CHUNK), jnp.int32),
        pltpu.VMEM((NCHUNK, CHUNK), jnp.int32),
        pltpu.VMEM((2, CHUNK, 128), jnp.float32),
        pltpu.VMEM((2, CHUNK, 128), jnp.float32),
        pltpu.VMEM((2, CHUNK, 128), jnp.float32),
        pltpu.VMEM((BPW,), jnp.float32),
        pltpu.SemaphoreType.DMA,
        pltpu.SemaphoreType.DMA,
        pltpu.SemaphoreType.DMA,
    ],
)(_mirt_body)


def kernel(user, item, theta_table, a_table, b_table):
    user = user.astype(jnp.int32).reshape(NW, BPW)
    item = item.astype(jnp.int32).reshape(NW, BPW)
    theta128 = theta_table.reshape(theta_table.shape[0] // GROUP, 128)
    a128 = a_table.reshape(a_table.shape[0] // GROUP, 128)
    b128 = jnp.pad(b_table.reshape(-1), (0, BROW * 128 - b_table.shape[0]))
    b128 = b128.reshape(BROW, 128)
    out = _mirt(user, item, theta128, a128, b128)
    return out.reshape(B)


# speedup vs baseline: 3.9184x; 3.9184x over previous
import functools

import jax
import jax.numpy as jnp
from jax import lax
from jax.experimental import pallas as pl
from jax.experimental.pallas import tpu as pltpu
from jax.experimental.pallas import tpu_sc as plsc

B = 16384
D = 16
NC = 2
NS = 16
NW = NC * NS
BPW = B // NW
CHUNK = 128
NCHUNK = BPW // CHUNK
GBUF = 16

_LOG1P_COEFS = (
    5.2394028005e-09,
    9.9999891058e-01,
    -4.9996224452e-01,
    3.3281842540e-01,
    -2.4635660618e-01,
    1.8468848463e-01,
    -1.2526661441e-01,
    6.6512479382e-02,
    -2.3038279977e-02,
    3.7526242255e-03,
)


def _softplus(x):
    t = jnp.exp(-jnp.abs(x))
    p = jnp.full((16,), _LOG1P_COEFS[-1], jnp.float32)
    for c in reversed(_LOG1P_COEFS[:-1]):
        p = p * t + jnp.float32(c)
    return jnp.maximum(x, jnp.float32(0.0)) + p


def _theta_body(user_hbm, theta_hbm, out_hbm,
                uidx8_v, blk_v, rows_v, sem):
    wid = lax.axis_index("s") * NC + lax.axis_index("c")
    w8 = lax.shift_right_logical(wid, 3) * 8
    r8 = jnp.bitwise_and(wid, 7)
    pltpu.sync_copy(
        user_hbm.at[pl.ds(pl.multiple_of(w8, 8), 8), :], uidx8_v)

    iota16 = lax.iota(jnp.int32, 16)
    r8vec = jnp.full((16,), r8, jnp.int32)

    def step(g, carry):
        base = g * GBUF
        u16 = plsc.load_gather(uidx8_v, [r8vec, base + iota16])
        grp16 = lax.shift_right_logical(u16, 7)
        lane16 = jnp.bitwise_and(u16, 127)
        for t in range(GBUF):
            start = grp16[t] * 128
            pltpu.async_copy(
                theta_hbm.at[:, pl.ds(pl.multiple_of(start, 128), 128)],
                blk_v.at[t], sem)
        for t in range(GBUF):
            pltpu.make_async_copy(theta_hbm.at[:, pl.ds(0, 128)],
                                  blk_v.at[t], sem).wait()
        for t in range(GBUF):
            tvec = jnp.full((16,), t, jnp.int32)
            lvec = jnp.full((16,), lane16[t], jnp.int32)
            col = plsc.load_gather(blk_v, [tvec, iota16, lvec])
            rows_v[pl.ds((base + t) * D, D)] = col
        return carry

    lax.fori_loop(0, BPW // GBUF, step, 0)

    pltpu.sync_copy(rows_v, out_hbm.at[pl.ds(wid * (BPW * D), BPW * D)])


_theta_gather = functools.partial(
    pl.kernel,
    out_type=jax.ShapeDtypeStruct((B * D,), jnp.float32),
    mesh=plsc.VectorSubcoreMesh(core_axis_name="c", subcore_axis_name="s"),
    compiler_params=pltpu.CompilerParams(
        needs_layout_passes=False, use_tc_tiling_on_sc=True),
    scratch_types=[
        pltpu.VMEM((8, BPW), jnp.int32),
        pltpu.VMEM((GBUF, D, 128), jnp.float32),
        pltpu.VMEM((BPW * D,), jnp.float32),
        pltpu.SemaphoreType.DMA,
    ],
)(_theta_body)


def _mirt_body(item_hbm, thg_hbm, a_hbm, b_hbm, out_hbm,
               iidx_v, th_v, a_v, b_v, out_v, sem_t, sem_a, sem_b):
    wid = lax.axis_index("s") * NC + lax.axis_index("c")

    pltpu.sync_copy(item_hbm.at[wid], iidx_v)
    cp_th = pltpu.async_copy(
        thg_hbm.at[pl.ds(wid * (BPW * D), BPW * D)], th_v, sem_t)

    copies = []
    for k in range(NCHUNK):
        copies.append(pltpu.async_copy(a_hbm.at[iidx_v.at[k]],
                                       a_v.at[k], sem_a))
        copies.append(pltpu.async_copy(b_hbm.at[iidx_v.at[k]],
                                       b_v.at[k], sem_b))
    cp_th.wait()
    for cp in copies:
        cp.wait()

    iota16 = lax.iota(jnp.int32, 16)

    for k in range(NCHUNK):
        kvec = jnp.full((16,), k, jnp.int32)

        def block(j, carry, k=k, kvec=kvec):
            rows = j * 16 + iota16
            flat0 = (k * CHUNK + rows) * D
            acc = -plsc.load_gather(b_v, [kvec, rows])
            for d in range(D):
                dvec = jnp.full((16,), d, jnp.int32)
                th = plsc.load_gather(th_v, [flat0 + d])
                av = plsc.load_gather(a_v, [kvec, rows, dvec])
                acc = acc + _softplus(av) * th
            res = jnp.float32(1.0) / (jnp.float32(1.0) + jnp.exp(-acc))
            out_v[pl.ds(k * CHUNK + j * 16, 16)] = res
            return carry

        lax.fori_loop(0, CHUNK // 16, block, 0)

    pltpu.sync_copy(out_v, out_hbm.at[wid])


_mirt = functools.partial(
    pl.kernel,
    out_type=jax.ShapeDtypeStruct((NW, BPW), jnp.float32),
    mesh=plsc.VectorSubcoreMesh(core_axis_name="c", subcore_axis_name="s"),
    compiler_params=pltpu.CompilerParams(
        needs_layout_passes=False, use_tc_tiling_on_sc=False),
    scratch_types=[
        pltpu.VMEM((NCHUNK, CHUNK), jnp.int32),
        pltpu.VMEM((BPW * D,), jnp.float32),
        pltpu.VMEM((NCHUNK, CHUNK, D), jnp.float32),
        pltpu.VMEM((NCHUNK, CHUNK), jnp.float32),
        pltpu.VMEM((BPW,), jnp.float32),
        pltpu.SemaphoreType.DMA,
        pltpu.SemaphoreType.DMA,
        pltpu.SemaphoreType.DMA,
    ],
)(_mirt_body)


def kernel(user, item, theta_table, a_table, b_table):
    user = user.astype(jnp.int32).reshape(NW, BPW)
    item = item.astype(jnp.int32).reshape(NW, NCHUNK, CHUNK)
    theta_t = theta_table.T
    b_flat = b_table.reshape(-1)
    thg = _theta_gather(user, theta_t)
    out = _mirt(item, thg, a_table, b_flat)
    return out.reshape(B)


# speedup vs baseline: 4.5833x; 1.1697x over previous
import functools

import jax
import jax.numpy as jnp
from jax import lax
from jax.experimental import pallas as pl
from jax.experimental.pallas import tpu as pltpu
from jax.experimental.pallas import tpu_sc as plsc

B = 16384
D = 16
NC = 2
NS = 16
NW = NC * NS
BPW = B // NW
CHUNK = 128
NCHUNK = BPW // CHUNK
GBUF = 16

_LOG1P_COEFS = (
    5.2394028005e-09,
    9.9999891058e-01,
    -4.9996224452e-01,
    3.3281842540e-01,
    -2.4635660618e-01,
    1.8468848463e-01,
    -1.2526661441e-01,
    6.6512479382e-02,
    -2.3038279977e-02,
    3.7526242255e-03,
)


def _softplus(x):
    t = jnp.exp(-jnp.abs(x))
    p = jnp.full((16,), _LOG1P_COEFS[-1], jnp.float32)
    for c in reversed(_LOG1P_COEFS[:-1]):
        p = p * t + jnp.float32(c)
    return jnp.maximum(x, jnp.float32(0.0)) + p


def _theta_body(user_hbm, theta_hbm, out_hbm,
                uidx8_v, blk_v, rows_v, sem):
    wid = lax.axis_index("s") * NC + lax.axis_index("c")
    w8 = lax.shift_right_logical(wid, 3) * 8
    r8 = jnp.bitwise_and(wid, 7)
    pltpu.sync_copy(
        user_hbm.at[pl.ds(pl.multiple_of(w8, 8), 8), :], uidx8_v)

    iota16 = lax.iota(jnp.int32, 16)
    r8vec = jnp.full((16,), r8, jnp.int32)
    NG = BPW // GBUF

    def fire(g, slot):
        u16 = plsc.load_gather(uidx8_v, [r8vec, g * GBUF + iota16])
        grp16 = lax.shift_right_logical(u16, 7)
        for t in range(GBUF):
            start = grp16[t] * 128
            pltpu.async_copy(
                theta_hbm.at[:, pl.ds(pl.multiple_of(start, 128), 128)],
                blk_v.at[slot, t], sem)
        return u16

    fire(0, 0)

    def step(g, carry):
        s = jnp.bitwise_and(g, 1)
        u16 = plsc.load_gather(uidx8_v, [r8vec, g * GBUF + iota16])
        lane16 = jnp.bitwise_and(u16, 127)

        @pl.when(g + 1 < NG)
        def _fire_next():
            fire(g + 1, 1 - s)

        for t in range(GBUF):
            pltpu.make_async_copy(theta_hbm.at[:, pl.ds(0, 128)],
                                  blk_v.at[0, t], sem).wait()
        svec = jnp.full((16,), s, jnp.int32)
        for t in range(GBUF):
            tvec = jnp.full((16,), t, jnp.int32)
            lvec = jnp.full((16,), lane16[t], jnp.int32)
            col = plsc.load_gather(blk_v, [svec, tvec, iota16, lvec])
            rows_v[pl.ds((g * GBUF + t) * D, D)] = col
        return carry

    lax.fori_loop(0, NG, step, 0)

    pltpu.sync_copy(rows_v, out_hbm.at[pl.ds(wid * (BPW * D), BPW * D)])


_theta_gather = functools.partial(
    pl.kernel,
    out_type=jax.ShapeDtypeStruct((B * D,), jnp.float32),
    mesh=plsc.VectorSubcoreMesh(core_axis_name="c", subcore_axis_name="s"),
    compiler_params=pltpu.CompilerParams(
        needs_layout_passes=False, use_tc_tiling_on_sc=True),
    scratch_types=[
        pltpu.VMEM((8, BPW), jnp.int32),
        pltpu.VMEM((2, GBUF, D, 128), jnp.float32),
        pltpu.VMEM((BPW * D,), jnp.float32),
        pltpu.SemaphoreType.DMA,
    ],
)(_theta_body)


def _mirt_body(item_hbm, thg_hbm, a_hbm, b_hbm, out_hbm,
               iidx_v, th_v, a_v, b_v, out_v, sem_t, sem_a, sem_b):
    wid = lax.axis_index("s") * NC + lax.axis_index("c")

    pltpu.sync_copy(item_hbm.at[wid], iidx_v)
    cp_th = pltpu.async_copy(
        thg_hbm.at[pl.ds(wid * (BPW * D), BPW * D)], th_v, sem_t)

    copies = []
    for k in range(NCHUNK):
        copies.append(pltpu.async_copy(a_hbm.at[iidx_v.at[k]],
                                       a_v.at[k], sem_a))
        copies.append(pltpu.async_copy(b_hbm.at[iidx_v.at[k]],
                                       b_v.at[k], sem_b))
    cp_th.wait()
    for cp in copies:
        cp.wait()

    iota16 = lax.iota(jnp.int32, 16)

    for k in range(NCHUNK):
        kvec = jnp.full((16,), k, jnp.int32)

        def block(j, carry, k=k, kvec=kvec):
            rows = j * 16 + iota16
            flat0 = (k * CHUNK + rows) * D
            acc = -plsc.load_gather(b_v, [kvec, rows])
            for d in range(D):
                dvec = jnp.full((16,), d, jnp.int32)
                th = plsc.load_gather(th_v, [flat0 + d])
                av = plsc.load_gather(a_v, [kvec, rows, dvec])
                acc = acc + _softplus(av) * th
            res = jnp.float32(1.0) / (jnp.float32(1.0) + jnp.exp(-acc))
            out_v[pl.ds(k * CHUNK + j * 16, 16)] = res
            return carry

        lax.fori_loop(0, CHUNK // 16, block, 0)

    pltpu.sync_copy(out_v, out_hbm.at[wid])


_mirt = functools.partial(
    pl.kernel,
    out_type=jax.ShapeDtypeStruct((NW, BPW), jnp.float32),
    mesh=plsc.VectorSubcoreMesh(core_axis_name="c", subcore_axis_name="s"),
    compiler_params=pltpu.CompilerParams(
        needs_layout_passes=False, use_tc_tiling_on_sc=False),
    scratch_types=[
        pltpu.VMEM((NCHUNK, CHUNK), jnp.int32),
        pltpu.VMEM((BPW * D,), jnp.float32),
        pltpu.VMEM((NCHUNK, CHUNK, D), jnp.float32),
        pltpu.VMEM((NCHUNK, CHUNK), jnp.float32),
        pltpu.VMEM((BPW,), jnp.float32),
        pltpu.SemaphoreType.DMA,
        pltpu.SemaphoreType.DMA,
        pltpu.SemaphoreType.DMA,
    ],
)(_mirt_body)


def kernel(user, item, theta_table, a_table, b_table):
    user = user.astype(jnp.int32).reshape(NW, BPW)
    item = item.astype(jnp.int32).reshape(NW, NCHUNK, CHUNK)
    theta_t = theta_table.T
    b_flat = b_table.reshape(-1)
    thg = _theta_gather(user, theta_t)
    out = _mirt(item, thg, a_table, b_flat)
    return out.reshape(B)


# speedup vs baseline: 4.6307x; 1.0103x over previous
import functools

import jax
import jax.numpy as jnp
from jax import lax
from jax.experimental import pallas as pl
from jax.experimental.pallas import tpu as pltpu
from jax.experimental.pallas import tpu_sc as plsc

B = 16384
D = 16
NC = 2
NS = 16
NW = NC * NS
BPW = B // NW
CHUNK = 128
NCHUNK = BPW // CHUNK
GBUF = 16

_LOG1P_COEFS = (
    1.4720650109e-06,
    9.9984769750e-01,
    -4.9737321616e-01,
    3.1574731676e-01,
    -1.9035433673e-01,
    8.2691237111e-02,
    -1.7414077524e-02,
)


def _softplus(x):
    t = jnp.exp(-jnp.abs(x))
    p = jnp.full((16,), _LOG1P_COEFS[-1], jnp.float32)
    for c in reversed(_LOG1P_COEFS[:-1]):
        p = p * t + jnp.float32(c)
    return jnp.maximum(x, jnp.float32(0.0)) + p


def _theta_body(user_hbm, theta_hbm, out_hbm,
                uidx8_v, blk_v, rows_v, sem):
    wid = lax.axis_index("s") * NC + lax.axis_index("c")
    w8 = lax.shift_right_logical(wid, 3) * 8
    r8 = jnp.bitwise_and(wid, 7)
    pltpu.sync_copy(
        user_hbm.at[pl.ds(pl.multiple_of(w8, 8), 8), :], uidx8_v)

    iota16 = lax.iota(jnp.int32, 16)
    r8vec = jnp.full((16,), r8, jnp.int32)
    NG = BPW // GBUF

    def fire(g, slot):
        u16 = plsc.load_gather(uidx8_v, [r8vec, g * GBUF + iota16])
        grp16 = lax.shift_right_logical(u16, 7)
        for t in range(GBUF):
            start = grp16[t] * 128
            pltpu.async_copy(
                theta_hbm.at[:, pl.ds(pl.multiple_of(start, 128), 128)],
                blk_v.at[slot, t], sem)
        return u16

    fire(0, 0)

    def step(g, carry):
        s = jnp.bitwise_and(g, 1)
        u16 = plsc.load_gather(uidx8_v, [r8vec, g * GBUF + iota16])
        lane16 = jnp.bitwise_and(u16, 127)

        @pl.when(g + 1 < NG)
        def _fire_next():
            fire(g + 1, 1 - s)

        for t in range(GBUF):
            pltpu.make_async_copy(theta_hbm.at[:, pl.ds(0, 128)],
                                  blk_v.at[0, t], sem).wait()
        svec = jnp.full((16,), s, jnp.int32)
        for t in range(GBUF):
            tvec = jnp.full((16,), t, jnp.int32)
            lvec = jnp.full((16,), lane16[t], jnp.int32)
            col = plsc.load_gather(blk_v, [svec, tvec, iota16, lvec])
            rows_v[pl.ds((g * GBUF + t) * D, D)] = col
        return carry

    lax.fori_loop(0, NG, step, 0)

    pltpu.sync_copy(rows_v, out_hbm.at[pl.ds(wid * (BPW * D), BPW * D)])


_theta_gather = functools.partial(
    pl.kernel,
    out_type=jax.ShapeDtypeStruct((B * D,), jnp.float32),
    mesh=plsc.VectorSubcoreMesh(core_axis_name="c", subcore_axis_name="s"),
    compiler_params=pltpu.CompilerParams(
        needs_layout_passes=False, use_tc_tiling_on_sc=True,
        skip_device_barrier=True),
    scratch_types=[
        pltpu.VMEM((8, BPW), jnp.int32),
        pltpu.VMEM((2, GBUF, D, 128), jnp.float32),
        pltpu.VMEM((BPW * D,), jnp.float32),
        pltpu.SemaphoreType.DMA,
    ],
)(_theta_body)


def _mirt_body(item_hbm, thg_hbm, a_hbm, b_hbm, out_hbm,
               iidx_v, th_v, a_v, b_v, out_v, sem_t, sem_a, sem_b):
    wid = lax.axis_index("s") * NC + lax.axis_index("c")

    pltpu.sync_copy(item_hbm.at[wid], iidx_v)
    cp_th = pltpu.async_copy(
        thg_hbm.at[pl.ds(wid * (BPW * D), BPW * D)], th_v, sem_t)

    copies = []
    for k in range(NCHUNK):
        copies.append(pltpu.async_copy(a_hbm.at[iidx_v.at[k]],
                                       a_v.at[k], sem_a))
        copies.append(pltpu.async_copy(b_hbm.at[iidx_v.at[k]],
                                       b_v.at[k], sem_b))
    cp_th.wait()
    for cp in copies:
        cp.wait()

    iota16 = lax.iota(jnp.int32, 16)

    for k in range(NCHUNK):
        kvec = jnp.full((16,), k, jnp.int32)

        def block(j, carry, k=k, kvec=kvec):
            rows = j * 16 + iota16
            flat0 = (k * CHUNK + rows) * D
            acc = -plsc.load_gather(b_v, [kvec, rows])
            for d in range(D):
                dvec = jnp.full((16,), d, jnp.int32)
                th = plsc.load_gather(th_v, [flat0 + d])
                av = plsc.load_gather(a_v, [kvec, rows, dvec])
                acc = acc + _softplus(av) * th
            res = jnp.float32(1.0) / (jnp.float32(1.0) + jnp.exp(-acc))
            out_v[pl.ds(k * CHUNK + j * 16, 16)] = res
            return carry

        lax.fori_loop(0, CHUNK // 16, block, 0)

    pltpu.sync_copy(out_v, out_hbm.at[wid])


_mirt = functools.partial(
    pl.kernel,
    out_type=jax.ShapeDtypeStruct((NW, BPW), jnp.float32),
    mesh=plsc.VectorSubcoreMesh(core_axis_name="c", subcore_axis_name="s"),
    compiler_params=pltpu.CompilerParams(
        needs_layout_passes=False, use_tc_tiling_on_sc=False,
        skip_device_barrier=True),
    scratch_types=[
        pltpu.VMEM((NCHUNK, CHUNK), jnp.int32),
        pltpu.VMEM((BPW * D,), jnp.float32),
        pltpu.VMEM((NCHUNK, CHUNK, D), jnp.float32),
        pltpu.VMEM((NCHUNK, CHUNK), jnp.float32),
        pltpu.VMEM((BPW,), jnp.float32),
        pltpu.SemaphoreType.DMA,
        pltpu.SemaphoreType.DMA,
        pltpu.SemaphoreType.DMA,
    ],
)(_mirt_body)


def kernel(user, item, theta_table, a_table, b_table):
    user = user.astype(jnp.int32).reshape(NW, BPW)
    item = item.astype(jnp.int32).reshape(NW, NCHUNK, CHUNK)
    theta_t = theta_table.T
    b_flat = b_table.reshape(-1)
    thg = _theta_gather(user, theta_t)
    out = _mirt(item, thg, a_table, b_flat)
    return out.reshape(B)
